# baseline (speedup 1.0000x reference)
import functools

import jax
import jax.numpy as jnp
from jax import lax
from jax.experimental import pallas as pl
from jax.experimental.pallas import tpu as pltpu
from jax.experimental.pallas import tpu_sc as plsc

EMBED = 64
HIDDEN = 128
BATCH = 16384

_INFO = plsc.get_sparse_core_info()
_NC = _INFO.num_cores
_NS = _INFO.num_subcores
_NW = _NC * _NS
_BPW = BATCH // _NW
_CHUNK = 128
_NCHUNK = _BPW // _CHUNK


def _gather_body(table_hbm, idx_hbm, out_hbm, idx_v, rows_v, sem):
    wid = lax.axis_index("s") * _NC + lax.axis_index("c")
    pltpu.sync_copy(idx_hbm.at[pl.ds(wid * _NCHUNK, _NCHUNK)], idx_v)
    copies = [
        pltpu.async_copy(
            table_hbm.at[idx_v.at[j]],
            rows_v.at[pl.ds(j * _CHUNK, _CHUNK)],
            sem,
        )
        for j in range(_NCHUNK)
    ]
    for c in copies:
        c.wait()
    pltpu.sync_copy(rows_v, out_hbm.at[pl.ds(wid * _BPW, _BPW)])


_gather = pl.kernel(
    _gather_body,
    mesh=plsc.VectorSubcoreMesh(core_axis_name="c", subcore_axis_name="s"),
    out_type=jax.ShapeDtypeStruct((BATCH, EMBED), jnp.float32),
    scratch_types=[
        pltpu.VMEM((_NCHUNK, _CHUNK), jnp.int32),
        pltpu.VMEM((_BPW, EMBED), jnp.float32),
        pltpu.SemaphoreType.DMA,
    ],
    compiler_params=pltpu.CompilerParams(use_tc_tiling_on_sc=False),
)


_TB = 2048


def _mlp_body(x_ref, w1_ref, b1_ref, w2_ref, b2_ref, o_ref):
    h = jnp.dot(x_ref[...], w1_ref[...], preferred_element_type=jnp.float32)
    h = jnp.maximum(h + b1_ref[...], 0.0)
    o = jnp.dot(h, w2_ref[...], preferred_element_type=jnp.float32)
    o_ref[...] = o + b2_ref[...]


_mlp = pl.pallas_call(
    _mlp_body,
    grid=(BATCH // _TB,),
    in_specs=[
        pl.BlockSpec((_TB, EMBED), lambda i: (i, 0)),
        pl.BlockSpec((EMBED, HIDDEN), lambda i: (0, 0)),
        pl.BlockSpec((1, HIDDEN), lambda i: (0, 0)),
        pl.BlockSpec((HIDDEN, EMBED), lambda i: (0, 0)),
        pl.BlockSpec((1, EMBED), lambda i: (0, 0)),
    ],
    out_specs=pl.BlockSpec((_TB, EMBED), lambda i: (i, 0)),
    out_shape=jax.ShapeDtypeStruct((BATCH, EMBED), jnp.float32),
)


def kernel(user_id, table, W1, b1, W2, b2):
    idx = user_id.astype(jnp.int32).reshape(BATCH // _CHUNK, _CHUNK)
    x = _gather(table, idx)
    return _mlp(x, W1, b1.reshape(1, HIDDEN), W2, b2.reshape(1, EMBED))


# speedup vs baseline: 1.8086x; 1.8086x over previous
import jax
import jax.numpy as jnp
from jax import lax
from jax.experimental import pallas as pl
from jax.experimental.pallas import tpu as pltpu
from jax.experimental.pallas import tpu_sc as plsc

VOCAB1 = 100001
EMBED = 64
HIDDEN = 128
BATCH = 16384

_INFO = plsc.get_sparse_core_info()
_NC = _INFO.num_cores
_NS = _INFO.num_subcores
_NW = _NC * _NS
_DPW = EMBED // _NW
_CHUNK = 4096
_NCHUNK = BATCH // _CHUNK


def _gather_t_body(tableT_hbm, ids_hbm, outT_hbm, ids_v, row_v, out_v, sem):
    wid = lax.axis_index("s") * _NC + lax.axis_index("c")
    pltpu.sync_copy(ids_hbm, ids_v)
    for rr in range(_DPW):
        r = wid * _DPW + rr
        pltpu.sync_copy(tableT_hbm.at[r], row_v)
        for c in range(_NCHUNK):

            def body(k, carry, c=c):
                idx = ids_v[pl.ds(c * _CHUNK + k * 16, 16)]
                out_v[pl.ds(k * 16, 16)] = plsc.load_gather(row_v, [idx])
                return carry

            lax.fori_loop(0, _CHUNK // 16, body, 0)
            pltpu.sync_copy(out_v, outT_hbm.at[r, pl.ds(c * _CHUNK, _CHUNK)])


_gather_t = pl.kernel(
    _gather_t_body,
    mesh=plsc.VectorSubcoreMesh(core_axis_name="c", subcore_axis_name="s"),
    out_type=jax.ShapeDtypeStruct((EMBED, BATCH), jnp.float32),
    scratch_types=[
        pltpu.VMEM((BATCH,), jnp.int32),
        pltpu.VMEM((VOCAB1,), jnp.float32),
        pltpu.VMEM((_CHUNK,), jnp.float32),
        pltpu.SemaphoreType.DMA,
    ],
    compiler_params=pltpu.CompilerParams(needs_layout_passes=False),
)


_TB = 2048


def _mlp_t_body(xT_ref, w1_ref, b1_ref, w2_ref, b2_ref, oT_ref):
    xT = xT_ref[...]
    h = lax.dot_general(
        w1_ref[...], xT, (((0,), (0,)), ((), ())),
        preferred_element_type=jnp.float32,
    )
    h = jnp.maximum(h + b1_ref[...], 0.0)
    o = lax.dot_general(
        w2_ref[...], h, (((0,), (0,)), ((), ())),
        preferred_element_type=jnp.float32,
    )
    oT_ref[...] = o + b2_ref[...]


_mlp_t = pl.pallas_call(
    _mlp_t_body,
    grid=(BATCH // _TB,),
    in_specs=[
        pl.BlockSpec((EMBED, _TB), lambda i: (0, i)),
        pl.BlockSpec((EMBED, HIDDEN), lambda i: (0, 0)),
        pl.BlockSpec((HIDDEN, 1), lambda i: (0, 0)),
        pl.BlockSpec((HIDDEN, EMBED), lambda i: (0, 0)),
        pl.BlockSpec((EMBED, 1), lambda i: (0, 0)),
    ],
    out_specs=pl.BlockSpec((EMBED, _TB), lambda i: (0, i)),
    out_shape=jax.ShapeDtypeStruct((EMBED, BATCH), jnp.float32),
)


def kernel(user_id, table, W1, b1, W2, b2):
    xT = _gather_t(table.T, user_id.astype(jnp.int32))
    outT = _mlp_t(xT, W1, b1.reshape(HIDDEN, 1), W2, b2.reshape(EMBED, 1))
    return outT.T


# speedup vs baseline: 2.3608x; 1.3053x over previous
import jax
import jax.numpy as jnp
from jax import lax
from jax.experimental import pallas as pl
from jax.experimental.pallas import tpu as pltpu
from jax.experimental.pallas import tpu_sc as plsc

VOCAB1 = 100001
EMBED = 64
HIDDEN = 128
BATCH = 16384

_INFO = plsc.get_sparse_core_info()
_NC = _INFO.num_cores
_NS = _INFO.num_subcores
_NW = _NC * _NS
_DPW = EMBED // _NW
_CHUNK = 4096
_NCHUNK = BATCH // _CHUNK


def _gather_t_body(tableT_hbm, ids_hbm, outT_hbm, ids_v, row_v, out_v, sem):
    wid = lax.axis_index("s") * _NC + lax.axis_index("c")
    pltpu.sync_copy(ids_hbm, ids_v)
    pending = []
    for rr in range(_DPW):
        r = wid * _DPW + rr
        pltpu.sync_copy(tableT_hbm.at[r], row_v)
        for c in range(_NCHUNK):
            buf = (rr * _NCHUNK + c) % 2
            if len(pending) >= 2:
                pending.pop(0).wait()

            @plsc.parallel_loop(0, _CHUNK // 16, unroll=8)
            def body(k, c=c, buf=buf):
                idx = ids_v[pl.ds(c * _CHUNK + k * 16, 16)]
                out_v[buf, pl.ds(k * 16, 16)] = plsc.load_gather(row_v, [idx])

            pending.append(
                pltpu.async_copy(
                    out_v.at[buf], outT_hbm.at[r, pl.ds(c * _CHUNK, _CHUNK)], sem
                )
            )
    for p in pending:
        p.wait()


_gather_t = pl.kernel(
    _gather_t_body,
    mesh=plsc.VectorSubcoreMesh(core_axis_name="c", subcore_axis_name="s"),
    out_type=jax.ShapeDtypeStruct((EMBED, BATCH), jnp.float32),
    scratch_types=[
        pltpu.VMEM((BATCH,), jnp.int32),
        pltpu.VMEM((VOCAB1,), jnp.float32),
        pltpu.VMEM((2, _CHUNK), jnp.float32),
        pltpu.SemaphoreType.DMA,
    ],
    compiler_params=pltpu.CompilerParams(needs_layout_passes=False),
)


_TB = 2048


def _mlp_t_body(xT_ref, w1_ref, b1_ref, w2_ref, b2_ref, oT_ref):
    xT = xT_ref[...]
    h = lax.dot_general(
        w1_ref[...], xT, (((0,), (0,)), ((), ())),
        preferred_element_type=jnp.float32,
    )
    h = jnp.maximum(h + b1_ref[...], 0.0)
    o = lax.dot_general(
        w2_ref[...], h, (((0,), (0,)), ((), ())),
        preferred_element_type=jnp.float32,
    )
    oT_ref[...] = o + b2_ref[...]


_mlp_t = pl.pallas_call(
    _mlp_t_body,
    grid=(BATCH // _TB,),
    in_specs=[
        pl.BlockSpec((EMBED, _TB), lambda i: (0, i)),
        pl.BlockSpec((EMBED, HIDDEN), lambda i: (0, 0)),
        pl.BlockSpec((HIDDEN, 1), lambda i: (0, 0)),
        pl.BlockSpec((HIDDEN, EMBED), lambda i: (0, 0)),
        pl.BlockSpec((EMBED, 1), lambda i: (0, 0)),
    ],
    out_specs=pl.BlockSpec((EMBED, _TB), lambda i: (0, i)),
    out_shape=jax.ShapeDtypeStruct((EMBED, BATCH), jnp.float32),
)


def kernel(user_id, table, W1, b1, W2, b2):
    xT = _gather_t(table.T, user_id.astype(jnp.int32))
    outT = _mlp_t(xT, W1, b1.reshape(HIDDEN, 1), W2, b2.reshape(EMBED, 1))
    return outT.T


# speedup vs baseline: 2.3852x; 1.0103x over previous
import jax
import jax.numpy as jnp
from jax import lax
from jax.experimental import pallas as pl
from jax.experimental.pallas import tpu as pltpu
from jax.experimental.pallas import tpu_sc as plsc

VOCAB1 = 100001
EMBED = 64
HIDDEN = 128
BATCH = 16384

_INFO = plsc.get_sparse_core_info()
_NC = _INFO.num_cores
_NS = _INFO.num_subcores
_NW = _NC * _NS
_DPW = EMBED // _NW
_CHUNK = 4096
_NCHUNK = BATCH // _CHUNK


def _gather_t_body(tableT_hbm, ids_hbm, outT_hbm, ids_v, row_v, out_v, sem):
    wid = lax.axis_index("s") * _NC + lax.axis_index("c")
    pltpu.sync_copy(ids_hbm, ids_v)
    pending = []
    for rr in range(_DPW):
        r = wid * _DPW + rr
        pltpu.sync_copy(tableT_hbm.at[r], row_v)
        for c in range(_NCHUNK):
            buf = (rr * _NCHUNK + c) % 2
            if len(pending) >= 2:
                pending.pop(0).wait()

            @plsc.parallel_loop(0, _CHUNK // 16, unroll=8)
            def body(k, c=c, buf=buf):
                idx = ids_v[pl.ds(c * _CHUNK + k * 16, 16)]
                out_v[buf, pl.ds(k * 16, 16)] = plsc.load_gather(row_v, [idx])

            pending.append(
                pltpu.async_copy(
                    out_v.at[buf], outT_hbm.at[r, pl.ds(c * _CHUNK, _CHUNK)], sem
                )
            )
    for p in pending:
        p.wait()


_gather_t = pl.kernel(
    _gather_t_body,
    mesh=plsc.VectorSubcoreMesh(core_axis_name="c", subcore_axis_name="s"),
    out_type=jax.ShapeDtypeStruct((EMBED, BATCH), jnp.float32),
    scratch_types=[
        pltpu.VMEM((BATCH,), jnp.int32),
        pltpu.VMEM((VOCAB1,), jnp.float32),
        pltpu.VMEM((2, _CHUNK), jnp.float32),
        pltpu.SemaphoreType.DMA,
    ],
    compiler_params=pltpu.CompilerParams(
        needs_layout_passes=False, skip_device_barrier=True
    ),
)


_TB = 2048


def _mlp_t_body(xT_ref, w1_ref, b1_ref, w2_ref, b2_ref, oT_ref):
    xT = xT_ref[...]
    h = lax.dot_general(
        w1_ref[...], xT, (((0,), (0,)), ((), ())),
        preferred_element_type=jnp.float32,
    )
    h = jnp.maximum(h + b1_ref[...], 0.0)
    o = lax.dot_general(
        w2_ref[...], h, (((0,), (0,)), ((), ())),
        preferred_element_type=jnp.float32,
    )
    oT_ref[...] = o + b2_ref[...]


_mlp_t = pl.pallas_call(
    _mlp_t_body,
    grid=(BATCH // _TB,),
    in_specs=[
        pl.BlockSpec((EMBED, _TB), lambda i: (0, i)),
        pl.BlockSpec((EMBED, HIDDEN), lambda i: (0, 0)),
        pl.BlockSpec((HIDDEN, 1), lambda i: (0, 0)),
        pl.BlockSpec((HIDDEN, EMBED), lambda i: (0, 0)),
        pl.BlockSpec((EMBED, 1), lambda i: (0, 0)),
    ],
    out_specs=pl.BlockSpec((EMBED, _TB), lambda i: (0, i)),
    out_shape=jax.ShapeDtypeStruct((EMBED, BATCH), jnp.float32),
)


def kernel(user_id, table, W1, b1, W2, b2):
    xT = _gather_t(table.T, user_id.astype(jnp.int32))
    outT = _mlp_t(xT, W1, b1.reshape(HIDDEN, 1), W2, b2.reshape(EMBED, 1))
    return outT.T


# speedup vs baseline: 2.5260x; 1.0590x over previous
import jax
import jax.numpy as jnp
from jax import lax
from jax.experimental import pallas as pl
from jax.experimental.pallas import tpu as pltpu
from jax.experimental.pallas import tpu_sc as plsc

VOCAB1 = 100001
EMBED = 64
HIDDEN = 128
BATCH = 16384

_INFO = plsc.get_sparse_core_info()
_NC = _INFO.num_cores
_NS = _INFO.num_subcores
_NW = _NC * _NS
_DPW = EMBED // _NW
_CHUNK = 4096
_NCHUNK = BATCH // _CHUNK


def _gather_t_body(tableT_hbm, ids_hbm, outT_hbm, ids_v, row_v, out_v, sem):
    wid = lax.axis_index("s") * _NC + lax.axis_index("c")
    pltpu.sync_copy(ids_hbm, ids_v)
    pending = []
    for rr in range(_DPW):
        r = wid * _DPW + rr
        pltpu.sync_copy(tableT_hbm.at[r], row_v)
        for c in range(_NCHUNK):
            buf = (rr * _NCHUNK + c) % 2
            if len(pending) >= 2:
                pending.pop(0).wait()

            @plsc.parallel_loop(0, _CHUNK // 16, unroll=8)
            def body(k, c=c, buf=buf):
                idx = ids_v[pl.ds(c * _CHUNK + k * 16, 16)]
                out_v[buf, pl.ds(k * 16, 16)] = plsc.load_gather(row_v, [idx])

            pending.append(
                pltpu.async_copy(
                    out_v.at[buf], outT_hbm.at[r, pl.ds(c * _CHUNK, _CHUNK)], sem
                )
            )
    for p in pending:
        p.wait()


_gather_t = pl.kernel(
    _gather_t_body,
    mesh=plsc.VectorSubcoreMesh(core_axis_name="c", subcore_axis_name="s"),
    out_type=jax.ShapeDtypeStruct((EMBED, BATCH), jnp.float32),
    scratch_types=[
        pltpu.VMEM((BATCH,), jnp.int32),
        pltpu.VMEM((VOCAB1,), jnp.float32),
        pltpu.VMEM((2, _CHUNK), jnp.float32),
        pltpu.SemaphoreType.DMA,
    ],
    compiler_params=pltpu.CompilerParams(
        needs_layout_passes=False, skip_device_barrier=True
    ),
)


_TB = 4096


def _mlp_t_body(xT_ref, w1_ref, b1_ref, w2_ref, b2_ref, oT_ref):
    xT = xT_ref[...]
    h = lax.dot_general(
        w1_ref[...], xT, (((0,), (0,)), ((), ())),
        preferred_element_type=jnp.float32,
    )
    h = jnp.maximum(h + b1_ref[...], 0.0)
    o = lax.dot_general(
        w2_ref[...], h, (((0,), (0,)), ((), ())),
        preferred_element_type=jnp.float32,
    )
    oT_ref[...] = o + b2_ref[...]


_mlp_t = pl.pallas_call(
    _mlp_t_body,
    grid=(BATCH // _TB,),
    in_specs=[
        pl.BlockSpec((EMBED, _TB), lambda i: (0, i)),
        pl.BlockSpec((EMBED, HIDDEN), lambda i: (0, 0)),
        pl.BlockSpec((HIDDEN, 1), lambda i: (0, 0)),
        pl.BlockSpec((HIDDEN, EMBED), lambda i: (0, 0)),
        pl.BlockSpec((EMBED, 1), lambda i: (0, 0)),
    ],
    out_specs=pl.BlockSpec((EMBED, _TB), lambda i: (0, i)),
    out_shape=jax.ShapeDtypeStruct((EMBED, BATCH), jnp.float32),
)


def kernel(user_id, table, W1, b1, W2, b2):
    xT = _gather_t(table.T, user_id.astype(jnp.int32))
    outT = _mlp_t(xT, W1, b1.reshape(HIDDEN, 1), W2, b2.reshape(EMBED, 1))
    return outT.T


# speedup vs baseline: 2.5897x; 1.0252x over previous
import jax
import jax.numpy as jnp
from jax import lax
from jax.experimental import pallas as pl
from jax.experimental.pallas import tpu as pltpu
from jax.experimental.pallas import tpu_sc as plsc

VOCAB1 = 100001
EMBED = 64
HIDDEN = 128
BATCH = 16384

_INFO = plsc.get_sparse_core_info()
_NC = _INFO.num_cores
_NS = _INFO.num_subcores
_NW = _NC * _NS
_DPW = EMBED // _NW
_CHUNK = 4096
_NCHUNK = BATCH // _CHUNK


def _gather_t_body(tableT_hbm, ids_hbm, outT_hbm, ids_v, row_v, out_v, sem):
    wid = lax.axis_index("s") * _NC + lax.axis_index("c")
    pltpu.sync_copy(ids_hbm, ids_v)
    pending = []
    for rr in range(_DPW):
        r = wid * _DPW + rr
        pltpu.sync_copy(tableT_hbm.at[r], row_v)
        for c in range(_NCHUNK):
            buf = (rr * _NCHUNK + c) % 2
            if len(pending) >= 2:
                pending.pop(0).wait()

            @plsc.parallel_loop(0, _CHUNK // 16, unroll=8)
            def body(k, c=c, buf=buf):
                idx = ids_v[pl.ds(c * _CHUNK + k * 16, 16)]
                out_v[buf, pl.ds(k * 16, 16)] = plsc.load_gather(row_v, [idx])

            pending.append(
                pltpu.async_copy(
                    out_v.at[buf], outT_hbm.at[r, pl.ds(c * _CHUNK, _CHUNK)], sem
                )
            )
    for p in pending:
        p.wait()


_gather_t = pl.kernel(
    _gather_t_body,
    mesh=plsc.VectorSubcoreMesh(core_axis_name="c", subcore_axis_name="s"),
    out_type=jax.ShapeDtypeStruct((EMBED, BATCH), jnp.float32),
    scratch_types=[
        pltpu.VMEM((BATCH,), jnp.int32),
        pltpu.VMEM((VOCAB1,), jnp.float32),
        pltpu.VMEM((2, _CHUNK), jnp.float32),
        pltpu.SemaphoreType.DMA,
    ],
    compiler_params=pltpu.CompilerParams(
        needs_layout_passes=False, skip_device_barrier=True
    ),
)


_TB = 8192


def _mlp_t_body(xT_ref, w1_ref, b1_ref, w2_ref, b2_ref, oT_ref):
    xT = xT_ref[...]
    h = lax.dot_general(
        w1_ref[...], xT, (((0,), (0,)), ((), ())),
        preferred_element_type=jnp.float32,
    )
    h = jnp.maximum(h + b1_ref[...], 0.0)
    o = lax.dot_general(
        w2_ref[...], h, (((0,), (0,)), ((), ())),
        preferred_element_type=jnp.float32,
    )
    oT_ref[...] = o + b2_ref[...]


_mlp_t = pl.pallas_call(
    _mlp_t_body,
    grid=(BATCH // _TB,),
    in_specs=[
        pl.BlockSpec((EMBED, _TB), lambda i: (0, i)),
        pl.BlockSpec((EMBED, HIDDEN), lambda i: (0, 0)),
        pl.BlockSpec((HIDDEN, 1), lambda i: (0, 0)),
        pl.BlockSpec((HIDDEN, EMBED), lambda i: (0, 0)),
        pl.BlockSpec((EMBED, 1), lambda i: (0, 0)),
    ],
    out_specs=pl.BlockSpec((EMBED, _TB), lambda i: (0, i)),
    out_shape=jax.ShapeDtypeStruct((EMBED, BATCH), jnp.float32),
)


def kernel(user_id, table, W1, b1, W2, b2):
    xT = _gather_t(table.T, user_id.astype(jnp.int32))
    outT = _mlp_t(xT, W1, b1.reshape(HIDDEN, 1), W2, b2.reshape(EMBED, 1))
    return outT.T
